# baseline (speedup 1.0000x reference)
import functools

import jax
import jax.numpy as jnp
from jax import lax
from jax.experimental import pallas as pl
from jax.experimental.pallas import tpu as pltpu
from jax.experimental.pallas import tpu_sc as plsc

N = 50000
D = 128
E = 400000

NC = 2
NS = 16
LANES = 16

EPT = 25600
EPAD = NS * EPT
STAGE_E = 2560
STAGE_BLOCKS = EPT // STAGE_E
NBLK = STAGE_E // 128

CHUNK = 10112
NPASS = 5
ACC_ROWS = CHUNK + 128
ZPT = ACC_ROWS // NS
NROW_PAD = NPASS * CHUNK

NP = N + 16


def _sc_mesh():
  return plsc.VectorSubcoreMesh(core_axis_name="c", subcore_axis_name="s")


EPT_R = E // NS
_DEG_FULL = EPT_R // LANES


def _deg_body(e0_hbm, e1_hbm, out_hbm, idx_v, hist_v):
  c = lax.axis_index("c")
  s = lax.axis_index("s")
  ones16 = jnp.ones((LANES,), jnp.float32)
  tailm = lax.iota(jnp.int32, LANES) < (EPT_R - _DEG_FULL * LANES)

  def process(e_ref):
    for kind in range(2):
      def zero_body(i, _):
        hist_v[pl.ds(i * LANES, LANES)] = jnp.zeros((LANES,), jnp.float32)
        return 0

      lax.fori_loop(0, NP // LANES, zero_body, 0, unroll=8)

      pltpu.sync_copy(e_ref.at[pl.ds(kind * E + s * EPT_R, EPT_R)],
                      idx_v.at[pl.ds(0, EPT_R)])

      def hist_body(i, _):
        v = idx_v[pl.ds(i * LANES, LANES)]
        plsc.addupdate_scatter(hist_v, [v], ones16)
        return 0

      lax.fori_loop(0, _DEG_FULL, hist_body, 0, unroll=8)
      vt = idx_v[pl.ds(_DEG_FULL * LANES, LANES)]
      plsc.addupdate_scatter(hist_v, [vt], ones16, mask=tailm)

      pltpu.sync_copy(hist_v, out_hbm.at[c, kind, s])

  @pl.when(c == 0)
  def _():
    process(e0_hbm)

  @pl.when(c == 1)
  def _():
    process(e1_hbm)


def _degree_kernel(e0, e1):
  return pl.kernel(
      _deg_body,
      out_type=jax.ShapeDtypeStruct((2, 2, NS, NP), jnp.float32),
      mesh=_sc_mesh(),
      scratch_types=[
          pltpu.VMEM((EPT_R + LANES,), jnp.int32),
          pltpu.VMEM((NP,), jnp.float32),
      ],
      compiler_params=pltpu.CompilerParams(needs_layout_passes=False),
  )(e0, e1)


_RB = 1024
_NRB = (N + _RB - 1) // _RB


def _mm_body(x_ref, degp_ref, w0_ref, w1_ref, y_ref, nd_ref):
  dp = degp_ref[...]
  deg = jnp.sum(dp, axis=2)
  norm = jnp.where(deg > 0, lax.rsqrt(jnp.maximum(deg, 1.0)), 0.0)
  xb = x_ref[...]
  y_ref[0] = jnp.dot(xb * norm[0, 0][:, None], w0_ref[...],
                     preferred_element_type=jnp.float32)
  y_ref[1] = jnp.dot(xb * norm[1, 0][:, None], w1_ref[...],
                     preferred_element_type=jnp.float32)
  nd_ref[0] = norm[0, 1]
  nd_ref[1] = norm[1, 1]


def _matmul_kernel(x, degp, w0, w1):
  return pl.pallas_call(
      _mm_body,
      grid=(_NRB,),
      in_specs=[
          pl.BlockSpec((_RB, D), lambda i: (i, 0)),
          pl.BlockSpec((2, 2, NS, _RB), lambda i: (0, 0, 0, i)),
          pl.BlockSpec((D, D), lambda i: (0, 0)),
          pl.BlockSpec((D, D), lambda i: (0, 0)),
      ],
      out_specs=[
          pl.BlockSpec((2, _RB, D), lambda i: (0, i, 0)),
          pl.BlockSpec((2, _RB), lambda i: (0, i)),
      ],
      out_shape=[
          jax.ShapeDtypeStruct((2, N, D), jnp.float32),
          jax.ShapeDtypeStruct((2, N), jnp.float32),
      ],
  )(x, degp, w0, w1)


def _scatter_body(y_hbm, edges_hbm, agg_hbm, acc, src_sa, dst_sa, src_sb,
                  dst_sb, srcflat, dstflat, rows_a, rows_b, rows_c, rows_d,
                  fill_smem, sem, sem_s, sem_d, sem_s0, sem_s1, sem_s2,
                  sem_s3):
  c = lax.axis_index("c")
  s = lax.axis_index("s")
  src_base = (c * 2 + 0) * EPAD + s * EPT
  dst_base = (c * 2 + 1) * EPAD + s * EPT
  yoff = c * N

  lane_iota = lax.iota(jnp.int32, LANES)
  pad_src = s * LANES + lane_iota + yoff
  pad_dst = CHUNK + lane_iota

  def pass_body(p, _):
    lo = pl.multiple_of(p * CHUNK, CHUNK)
    hi = jnp.minimum(lo + CHUNK, N)

    def zzero(i, _):
      j = i // 8
      k = i % 8
      rows_a[j, pl.ds(k * LANES, LANES)] = jnp.zeros((LANES,), jnp.float32)
      return 0

    lax.fori_loop(0, 64 * 8, zzero, 0, unroll=8)
    for k in range(10):
      pltpu.sync_copy(rows_a, acc.at[pl.ds(s * ZPT + k * 64, 64)])
    plsc.subcore_barrier()

    rows = (rows_a, rows_b, rows_c, rows_d)
    ssems = (sem_s0, sem_s1, sem_s2, sem_s3)
    QR = 64

    def gidx(j):
      return y_hbm.at[srcflat.at[pl.ds(j * QR, QR)]]

    def didx(j):
      return acc.at[dstflat.at[pl.ds(j * QR, QR)]]

    def drain(nb):
      @pl.when(nb > 0)
      def _():
        pltpu.async_copy(gidx(0), rows_a, sem)

      def drain_body(j, _):
        for b in range(4):
          bn = (b + 1) % 4

          @pl.when((j % 4) == b)
          def _():
            pltpu.make_async_copy(gidx(j), rows[b], sem).wait()

            @pl.when(j + 1 < nb)
            def _():
              @pl.when(j >= 3)
              def _():
                pltpu.make_async_copy(
                    rows[bn], acc.at[pl.ds(0, QR)], ssems[bn]).wait()

              pltpu.async_copy(gidx(j + 1), rows[bn], sem)

            pltpu.async_copy(rows[b], didx(j), ssems[b], add=True)

        return 0

      lax.fori_loop(0, nb, drain_body, 0)

      for b in range(4):
        @pl.when(nb > b)
        def _():
          pltpu.make_async_copy(
              rows[b], acc.at[pl.ds(0, QR)], ssems[b]).wait()

    def stage_start(b, src_s, dst_s):
      soff = pl.multiple_of(src_base + b * STAGE_E, STAGE_E)
      doff = pl.multiple_of(dst_base + b * STAGE_E, STAGE_E)
      pltpu.async_copy(edges_hbm.at[pl.ds(soff, STAGE_E)], src_s, sem_s)
      pltpu.async_copy(edges_hbm.at[pl.ds(doff, STAGE_E)], dst_s, sem_d)

    def stage_wait(src_s, dst_s):
      pltpu.make_async_copy(edges_hbm.at[pl.ds(0, STAGE_E)], src_s,
                            sem_s).wait()
      pltpu.make_async_copy(edges_hbm.at[pl.ds(0, STAGE_E)], dst_s,
                            sem_d).wait()

    stage_start(0, src_sa, dst_sa)

    def make_append(src_s, dst_s):
      def append_body(i, fill):
        sv = src_s[pl.ds(i * LANES, LANES)]
        dv = dst_s[pl.ds(i * LANES, LANES)]
        dloc = dv - lo
        m = dloc.astype(jnp.uint32) < (hi - lo).astype(jnp.uint32)
        plsc.store_compressed(srcflat.at[pl.ds(fill, LANES)], sv + yoff,
                              mask=m)
        plsc.store_compressed(dstflat.at[pl.ds(fill, LANES)], dloc,
                              mask=m)
        return fill + plsc.all_reduce_population_count(m)[0]
      return append_body

    def stage_body(b, fill0):
      def run(src_s, dst_s, src_n, dst_n):
        stage_wait(src_s, dst_s)

        @pl.when(b + 1 < STAGE_BLOCKS)
        def _():
          stage_start(b + 1, src_n, dst_n)

        fill_smem[0] = lax.fori_loop(0, STAGE_E // LANES,
                                     make_append(src_s, dst_s), fill0,
                                     unroll=4)

      beven = (b & 1) == 0

      @pl.when(beven)
      def _():
        run(src_sa, dst_sa, src_sb, dst_sb)

      @pl.when(jnp.logical_not(beven))
      def _():
        run(src_sb, dst_sb, src_sa, dst_sa)

      fill = fill_smem[0]
      nb = fill // QR
      drain(nb)

      @pl.when(nb > 0)
      def _():
        off = pl.multiple_of(nb * QR, QR)
        for k in range(4):
          srcflat[pl.ds(k * LANES, LANES)] = (
              srcflat[pl.ds(off + k * LANES, LANES)])
          dstflat[pl.ds(k * LANES, LANES)] = (
              dstflat[pl.ds(off + k * LANES, LANES)])

      return fill - nb * QR

    fill = lax.fori_loop(0, STAGE_BLOCKS, stage_body, 0)

    @pl.when(fill > 0)
    def _():
      f = fill
      npad = QR - f
      for k in range(4):
        cnt_k = jnp.clip(npad - k * LANES, 0, LANES)
        pm = lane_iota < cnt_k
        plsc.store_compressed(srcflat.at[pl.ds(f, LANES)], pad_src,
                              mask=pm)
        plsc.store_compressed(dstflat.at[pl.ds(f, LANES)], pad_dst,
                              mask=pm)
        f = f + cnt_k
      drain(1)

    plsc.subcore_barrier()

    rows_pt = CHUNK // NS
    foff = pl.multiple_of(c * NROW_PAD + lo + s * rows_pt, 8)
    pltpu.sync_copy(acc.at[pl.ds(s * rows_pt, rows_pt)],
                    agg_hbm.at[pl.ds(foff, rows_pt)])
    plsc.subcore_barrier()
    return 0

  lax.fori_loop(0, NPASS, pass_body, 0)


def _scatter_kernel(y2, edges1d):
  return pl.kernel(
      _scatter_body,
      out_type=jax.ShapeDtypeStruct((2 * NROW_PAD, D), jnp.float32),
      mesh=_sc_mesh(),
      scratch_types=[
          pltpu.VMEM_SHARED((ACC_ROWS, D), jnp.float32),
          pltpu.VMEM((STAGE_E,), jnp.int32),
          pltpu.VMEM((STAGE_E,), jnp.int32),
          pltpu.VMEM((STAGE_E,), jnp.int32),
          pltpu.VMEM((STAGE_E,), jnp.int32),
          pltpu.VMEM((STAGE_E + 256,), jnp.int32),
          pltpu.VMEM((STAGE_E + 256,), jnp.int32),
          pltpu.VMEM((64, D), jnp.float32),
          pltpu.VMEM((64, D), jnp.float32),
          pltpu.VMEM((64, D), jnp.float32),
          pltpu.VMEM((64, D), jnp.float32),
          pltpu.SMEM((1,), jnp.int32),
          pltpu.SemaphoreType.DMA,
          pltpu.SemaphoreType.DMA,
          pltpu.SemaphoreType.DMA,
          pltpu.SemaphoreType.DMA,
          pltpu.SemaphoreType.DMA,
          pltpu.SemaphoreType.DMA,
          pltpu.SemaphoreType.DMA,
      ],
      compiler_params=pltpu.CompilerParams(needs_layout_passes=False),
  )(y2, edges1d)


def _post_body(a0_ref, a1_ref, nd_ref, g_ref, b_ref, out_ref, sums_ref):
  t = pl.program_id(0)
  i = pl.program_id(1)
  nd = nd_ref[...]
  hb = (a0_ref[0] * nd[0][:, None] + a1_ref[0] * nd[1][:, None])

  @pl.when((t == 0) & (i == 0))
  def _():
    sums_ref[...] = jnp.zeros_like(sums_ref)

  @pl.when(t == 0)
  def _():
    rows = i * _RB + lax.broadcasted_iota(jnp.int32, (_RB, 1), 0)
    hv = jnp.where(rows < N, hb, 0.0)
    sums_ref[0] += jnp.sum(hv, axis=0)
    sums_ref[1] += jnp.sum(hv * hv, axis=0)

  @pl.when(t == 1)
  def _():
    mean = sums_ref[0] / N
    var = jnp.maximum(sums_ref[1] / N - mean * mean, 0.0)
    scale = g_ref[0] * lax.rsqrt(var + 1e-5)
    shift = b_ref[0] - mean * scale
    tv = hb * scale + shift
    nrm = jnp.sqrt(jnp.sum(tv * tv, axis=1, keepdims=True))
    out_ref[...] = tv / jnp.maximum(nrm, 1e-12)


def _post_kernel(agg3, nd, gamma, beta):
  return pl.pallas_call(
      _post_body,
      grid=(2, _NRB),
      in_specs=[
          pl.BlockSpec((1, _RB, D), lambda t, i: (0, i, 0)),
          pl.BlockSpec((1, _RB, D), lambda t, i: (1, i, 0)),
          pl.BlockSpec((2, _RB), lambda t, i: (0, i)),
          pl.BlockSpec((1, D), lambda t, i: (0, 0)),
          pl.BlockSpec((1, D), lambda t, i: (0, 0)),
      ],
      out_specs=pl.BlockSpec((_RB, D), lambda t, i: (i, 0)),
      out_shape=jax.ShapeDtypeStruct((N, D), jnp.float32),
      scratch_shapes=[pltpu.VMEM((2, D), jnp.float32)],
  )(agg3, agg3, nd, gamma, beta)


@jax.jit
def kernel(x, edge_index_rel0, edge_index_rel1, W0, W1, gamma, beta):
  e0 = edge_index_rel0.astype(jnp.int32)
  e1 = edge_index_rel1.astype(jnp.int32)
  pad = EPAD - E
  padvals = N + (jnp.arange(pad, dtype=jnp.int32) % LANES)
  parts = []
  for ei in (e0, e1):
    for kind in range(2):
      parts.append(jnp.concatenate([ei[kind], padvals]))
  edges1d = jnp.concatenate(parts)

  degp = _degree_kernel(e0.reshape(-1), e1.reshape(-1))
  y, nd = _matmul_kernel(x, degp, W0, W1)
  y2 = y.reshape(2 * N, D)
  agg2 = _scatter_kernel(y2, edges1d)
  return _post_kernel(agg2.reshape(2, NROW_PAD, D), nd,
                      gamma.reshape(1, D), beta.reshape(1, D))


# speedup vs baseline: 1.1490x; 1.1490x over previous
import functools

import jax
import jax.numpy as jnp
from jax import lax
from jax.experimental import pallas as pl
from jax.experimental.pallas import tpu as pltpu
from jax.experimental.pallas import tpu_sc as plsc

N = 50000
D = 128
E = 400000

NC = 2
NS = 16
LANES = 16

EPT = 25600
EPAD = NS * EPT
STAGE_E = 2560
STAGE_BLOCKS = EPT // STAGE_E
NBLK = STAGE_E // 128

CHUNK = 10112
NPASS = 5
ACC_ROWS = CHUNK + 128
ZPT = ACC_ROWS // NS
NROW_PAD = NPASS * CHUNK

NP = N + 16


def _sc_mesh():
  return plsc.VectorSubcoreMesh(core_axis_name="c", subcore_axis_name="s")


EPT_R = E // NS
_DEG_FULL = EPT_R // LANES


def _deg_body(e0_hbm, e1_hbm, out_hbm, idx_v, hist_v):
  c = lax.axis_index("c")
  s = lax.axis_index("s")
  ones16 = jnp.ones((LANES,), jnp.float32)
  tailm = lax.iota(jnp.int32, LANES) < (EPT_R - _DEG_FULL * LANES)

  def process(e_ref):
    for kind in range(2):
      def zero_body(i, _):
        hist_v[pl.ds(i * LANES, LANES)] = jnp.zeros((LANES,), jnp.float32)
        return 0

      lax.fori_loop(0, NP // LANES, zero_body, 0, unroll=8)

      pltpu.sync_copy(e_ref.at[pl.ds(kind * E + s * EPT_R, EPT_R)],
                      idx_v.at[pl.ds(0, EPT_R)])

      def hist_body(i, _):
        v = idx_v[pl.ds(i * LANES, LANES)]
        plsc.addupdate_scatter(hist_v, [v], ones16)
        return 0

      lax.fori_loop(0, _DEG_FULL, hist_body, 0, unroll=8)
      vt = idx_v[pl.ds(_DEG_FULL * LANES, LANES)]
      plsc.addupdate_scatter(hist_v, [vt], ones16, mask=tailm)

      pltpu.sync_copy(hist_v, out_hbm.at[c, kind, s])

  @pl.when(c == 0)
  def _():
    process(e0_hbm)

  @pl.when(c == 1)
  def _():
    process(e1_hbm)


def _degree_kernel(e0, e1):
  return pl.kernel(
      _deg_body,
      out_type=jax.ShapeDtypeStruct((2, 2, NS, NP), jnp.float32),
      mesh=_sc_mesh(),
      scratch_types=[
          pltpu.VMEM((EPT_R + LANES,), jnp.int32),
          pltpu.VMEM((NP,), jnp.float32),
      ],
      compiler_params=pltpu.CompilerParams(needs_layout_passes=False),
  )(e0, e1)


_RB = 1024
_NRB = (N + _RB - 1) // _RB


def _mm_body(x_ref, degp_ref, w0_ref, w1_ref, y_ref, nd_ref):
  dp = degp_ref[...]
  deg = jnp.sum(dp, axis=2)
  norm = jnp.where(deg > 0, lax.rsqrt(jnp.maximum(deg, 1.0)), 0.0)
  xb = x_ref[...]
  y_ref[0] = jnp.dot(xb * norm[0, 0][:, None], w0_ref[...],
                     preferred_element_type=jnp.float32)
  y_ref[1] = jnp.dot(xb * norm[1, 0][:, None], w1_ref[...],
                     preferred_element_type=jnp.float32)
  nd_ref[0] = norm[0, 1]
  nd_ref[1] = norm[1, 1]


def _matmul_kernel(x, degp, w0, w1):
  return pl.pallas_call(
      _mm_body,
      grid=(_NRB,),
      in_specs=[
          pl.BlockSpec((_RB, D), lambda i: (i, 0)),
          pl.BlockSpec((2, 2, NS, _RB), lambda i: (0, 0, 0, i)),
          pl.BlockSpec((D, D), lambda i: (0, 0)),
          pl.BlockSpec((D, D), lambda i: (0, 0)),
      ],
      out_specs=[
          pl.BlockSpec((2, _RB, D), lambda i: (0, i, 0)),
          pl.BlockSpec((2, _RB), lambda i: (0, i)),
      ],
      out_shape=[
          jax.ShapeDtypeStruct((2, N, D), jnp.float32),
          jax.ShapeDtypeStruct((2, N), jnp.float32),
      ],
  )(x, degp, w0, w1)


def _scatter_body(y_hbm, edges_hbm, agg_hbm, acc, src_sa, dst_sa, src_sb,
                  dst_sb, srcflat, dstflat, rows_a, rows_b, fill_smem, sem,
                  sem_s, sem_d):
  c = lax.axis_index("c")
  s = lax.axis_index("s")
  src_base = (c * 2 + 0) * EPAD + s * EPT
  dst_base = (c * 2 + 1) * EPAD + s * EPT
  yoff = c * N

  lane_iota = lax.iota(jnp.int32, LANES)
  pad_src = s * LANES + lane_iota + yoff
  pad_dst = CHUNK + lane_iota

  def pass_body(p, _):
    lo = pl.multiple_of(p * CHUNK, CHUNK)
    hi = jnp.minimum(lo + CHUNK, N)

    def zzero(i, _):
      j = i // 8
      k = i % 8
      rows_a[j, pl.ds(k * LANES, LANES)] = jnp.zeros((LANES,), jnp.float32)
      return 0

    lax.fori_loop(0, 128 * 8, zzero, 0, unroll=8)
    for k in range(5):
      pltpu.sync_copy(rows_a, acc.at[pl.ds(s * ZPT + k * 128, 128)])
    plsc.subcore_barrier()

    QR = 128

    def gidx(j):
      return y_hbm.at[srcflat.at[pl.ds(j * QR, QR)]]

    def didx(j):
      return acc.at[dstflat.at[pl.ds(j * QR, QR)]]

    def drain(nb):
      @pl.when(nb > 0)
      def _():
        pltpu.async_copy(gidx(0), rows_a, sem)

      def drain_body(j, _):
        jeven = (j & 1) == 0

        @pl.when(jeven)
        def _():
          pltpu.make_async_copy(gidx(j), rows_a, sem).wait()

          @pl.when(j + 1 < nb)
          def _():
            pltpu.async_copy(gidx(j + 1), rows_b, sem)

          pltpu.sync_copy(rows_a, didx(j), add=True)

        @pl.when(jnp.logical_not(jeven))
        def _():
          pltpu.make_async_copy(gidx(j), rows_b, sem).wait()

          @pl.when(j + 1 < nb)
          def _():
            pltpu.async_copy(gidx(j + 1), rows_a, sem)

          pltpu.sync_copy(rows_b, didx(j), add=True)

        return 0

      lax.fori_loop(0, nb, drain_body, 0)

    def stage_start(b, src_s, dst_s):
      soff = pl.multiple_of(src_base + b * STAGE_E, STAGE_E)
      doff = pl.multiple_of(dst_base + b * STAGE_E, STAGE_E)
      pltpu.async_copy(edges_hbm.at[pl.ds(soff, STAGE_E)], src_s, sem_s)
      pltpu.async_copy(edges_hbm.at[pl.ds(doff, STAGE_E)], dst_s, sem_d)

    def stage_wait(src_s, dst_s):
      pltpu.make_async_copy(edges_hbm.at[pl.ds(0, STAGE_E)], src_s,
                            sem_s).wait()
      pltpu.make_async_copy(edges_hbm.at[pl.ds(0, STAGE_E)], dst_s,
                            sem_d).wait()

    stage_start(0, src_sa, dst_sa)

    def make_append(src_s, dst_s):
      def append_body(i, fill):
        sv = src_s[pl.ds(i * LANES, LANES)]
        dv = dst_s[pl.ds(i * LANES, LANES)]
        dloc = dv - lo
        m = dloc.astype(jnp.uint32) < (hi - lo).astype(jnp.uint32)
        plsc.store_compressed(srcflat.at[pl.ds(fill, LANES)], sv + yoff,
                              mask=m)
        plsc.store_compressed(dstflat.at[pl.ds(fill, LANES)], dloc,
                              mask=m)
        return fill + plsc.all_reduce_population_count(m)[0]
      return append_body

    def stage_body(b, fill0):
      def run(src_s, dst_s, src_n, dst_n):
        stage_wait(src_s, dst_s)

        @pl.when(b + 1 < STAGE_BLOCKS)
        def _():
          stage_start(b + 1, src_n, dst_n)

        fill_smem[0] = lax.fori_loop(0, STAGE_E // LANES,
                                     make_append(src_s, dst_s), fill0,
                                     unroll=4)

      beven = (b & 1) == 0

      @pl.when(beven)
      def _():
        run(src_sa, dst_sa, src_sb, dst_sb)

      @pl.when(jnp.logical_not(beven))
      def _():
        run(src_sb, dst_sb, src_sa, dst_sa)

      fill = fill_smem[0]
      nb = fill // QR
      drain(nb)

      @pl.when(nb > 0)
      def _():
        off = pl.multiple_of(nb * QR, QR)
        for k in range(8):
          srcflat[pl.ds(k * LANES, LANES)] = (
              srcflat[pl.ds(off + k * LANES, LANES)])
          dstflat[pl.ds(k * LANES, LANES)] = (
              dstflat[pl.ds(off + k * LANES, LANES)])

      return fill - nb * QR

    fill = lax.fori_loop(0, STAGE_BLOCKS, stage_body, 0)

    @pl.when(fill > 0)
    def _():
      f = fill
      npad = QR - f
      for k in range(8):
        cnt_k = jnp.clip(npad - k * LANES, 0, LANES)
        pm = lane_iota < cnt_k
        plsc.store_compressed(srcflat.at[pl.ds(f, LANES)], pad_src,
                              mask=pm)
        plsc.store_compressed(dstflat.at[pl.ds(f, LANES)], pad_dst,
                              mask=pm)
        f = f + cnt_k
      drain(1)

    plsc.subcore_barrier()

    rows_pt = CHUNK // NS
    foff = pl.multiple_of(c * NROW_PAD + lo + s * rows_pt, 8)
    pltpu.sync_copy(acc.at[pl.ds(s * rows_pt, rows_pt)],
                    agg_hbm.at[pl.ds(foff, rows_pt)])
    plsc.subcore_barrier()
    return 0

  lax.fori_loop(0, NPASS, pass_body, 0)


def _scatter_kernel(y2, edges1d):
  return pl.kernel(
      _scatter_body,
      out_type=jax.ShapeDtypeStruct((2 * NROW_PAD, D), jnp.float32),
      mesh=_sc_mesh(),
      scratch_types=[
          pltpu.VMEM_SHARED((ACC_ROWS, D), jnp.float32),
          pltpu.VMEM((STAGE_E,), jnp.int32),
          pltpu.VMEM((STAGE_E,), jnp.int32),
          pltpu.VMEM((STAGE_E,), jnp.int32),
          pltpu.VMEM((STAGE_E,), jnp.int32),
          pltpu.VMEM((STAGE_E + 256,), jnp.int32),
          pltpu.VMEM((STAGE_E + 256,), jnp.int32),
          pltpu.VMEM((128, D), jnp.float32),
          pltpu.VMEM((128, D), jnp.float32),
          pltpu.SMEM((1,), jnp.int32),
          pltpu.SemaphoreType.DMA,
          pltpu.SemaphoreType.DMA,
          pltpu.SemaphoreType.DMA,
      ],
      compiler_params=pltpu.CompilerParams(needs_layout_passes=False),
  )(y2, edges1d)


def _post_body(a0_ref, a1_ref, nd_ref, g_ref, b_ref, out_ref, sums_ref):
  t = pl.program_id(0)
  i = pl.program_id(1)
  nd = nd_ref[...]
  hb = (a0_ref[0] * nd[0][:, None] + a1_ref[0] * nd[1][:, None])

  @pl.when((t == 0) & (i == 0))
  def _():
    sums_ref[...] = jnp.zeros_like(sums_ref)

  @pl.when(t == 0)
  def _():
    rows = i * _RB + lax.broadcasted_iota(jnp.int32, (_RB, 1), 0)
    hv = jnp.where(rows < N, hb, 0.0)
    sums_ref[0] += jnp.sum(hv, axis=0)
    sums_ref[1] += jnp.sum(hv * hv, axis=0)

  @pl.when(t == 1)
  def _():
    mean = sums_ref[0] / N
    var = jnp.maximum(sums_ref[1] / N - mean * mean, 0.0)
    scale = g_ref[0] * lax.rsqrt(var + 1e-5)
    shift = b_ref[0] - mean * scale
    tv = hb * scale + shift
    nrm = jnp.sqrt(jnp.sum(tv * tv, axis=1, keepdims=True))
    out_ref[...] = tv / jnp.maximum(nrm, 1e-12)


def _post_kernel(agg3, nd, gamma, beta):
  return pl.pallas_call(
      _post_body,
      grid=(2, _NRB),
      in_specs=[
          pl.BlockSpec((1, _RB, D), lambda t, i: (0, i, 0)),
          pl.BlockSpec((1, _RB, D), lambda t, i: (1, i, 0)),
          pl.BlockSpec((2, _RB), lambda t, i: (0, i)),
          pl.BlockSpec((1, D), lambda t, i: (0, 0)),
          pl.BlockSpec((1, D), lambda t, i: (0, 0)),
      ],
      out_specs=pl.BlockSpec((_RB, D), lambda t, i: (i, 0)),
      out_shape=jax.ShapeDtypeStruct((N, D), jnp.float32),
      scratch_shapes=[pltpu.VMEM((2, D), jnp.float32)],
  )(agg3, agg3, nd, gamma, beta)


@jax.jit
def kernel(x, edge_index_rel0, edge_index_rel1, W0, W1, gamma, beta):
  e0 = edge_index_rel0.astype(jnp.int32)
  e1 = edge_index_rel1.astype(jnp.int32)
  pad = EPAD - E
  padvals = N + (jnp.arange(pad, dtype=jnp.int32) % LANES)
  parts = []
  for ei in (e0, e1):
    for kind in range(2):
      parts.append(jnp.concatenate([ei[kind], padvals]))
  edges1d = jnp.concatenate(parts)

  degp = _degree_kernel(e0.reshape(-1), e1.reshape(-1))
  y, nd = _matmul_kernel(x, degp, W0, W1)
  y2 = y.reshape(2 * N, D)
  agg2 = _scatter_kernel(y2, edges1d)
  return _post_kernel(agg2.reshape(2, NROW_PAD, D), nd,
                      gamma.reshape(1, D), beta.reshape(1, D))


# speedup vs baseline: 1.1516x; 1.0023x over previous
import functools

import jax
import jax.numpy as jnp
from jax import lax
from jax.experimental import pallas as pl
from jax.experimental.pallas import tpu as pltpu
from jax.experimental.pallas import tpu_sc as plsc

N = 50000
D = 128
E = 400000

NC = 2
NS = 16
LANES = 16

EPT = 25600
EPAD = NS * EPT
STAGE_E = 2560
STAGE_BLOCKS = EPT // STAGE_E
NBLK = STAGE_E // 128

CHUNK = 10112
NPASS = 5
ACC_ROWS = CHUNK + 128
ZPT = ACC_ROWS // NS
NROW_PAD = NPASS * CHUNK

NP = N + 16


def _sc_mesh():
  return plsc.VectorSubcoreMesh(core_axis_name="c", subcore_axis_name="s")


EPT_R = E // NS
_DEG_FULL = EPT_R // LANES


def _deg_body(e0_hbm, e1_hbm, out_hbm, idx_v, hist_v):
  c = lax.axis_index("c")
  s = lax.axis_index("s")
  ones16 = jnp.ones((LANES,), jnp.float32)
  tailm = lax.iota(jnp.int32, LANES) < (EPT_R - _DEG_FULL * LANES)

  def process(e_ref):
    for kind in range(2):
      def zero_body(i, _):
        hist_v[pl.ds(i * LANES, LANES)] = jnp.zeros((LANES,), jnp.float32)
        return 0

      lax.fori_loop(0, NP // LANES, zero_body, 0, unroll=8)

      pltpu.sync_copy(e_ref.at[pl.ds(kind * E + s * EPT_R, EPT_R)],
                      idx_v.at[pl.ds(0, EPT_R)])

      def hist_body(i, _):
        v = idx_v[pl.ds(i * LANES, LANES)]
        plsc.addupdate_scatter(hist_v, [v], ones16)
        return 0

      lax.fori_loop(0, _DEG_FULL, hist_body, 0, unroll=8)
      vt = idx_v[pl.ds(_DEG_FULL * LANES, LANES)]
      plsc.addupdate_scatter(hist_v, [vt], ones16, mask=tailm)

      pltpu.sync_copy(hist_v, out_hbm.at[c, kind, s])

  @pl.when(c == 0)
  def _():
    process(e0_hbm)

  @pl.when(c == 1)
  def _():
    process(e1_hbm)


def _degree_kernel(e0, e1):
  return pl.kernel(
      _deg_body,
      out_type=jax.ShapeDtypeStruct((2, 2, NS, NP), jnp.float32),
      mesh=_sc_mesh(),
      scratch_types=[
          pltpu.VMEM((EPT_R + LANES,), jnp.int32),
          pltpu.VMEM((NP,), jnp.float32),
      ],
      compiler_params=pltpu.CompilerParams(needs_layout_passes=False),
  )(e0, e1)


_RB = 1024
_NRB = (N + _RB - 1) // _RB


def _mm_body(x_ref, degp_ref, w0_ref, w1_ref, y_ref, nd_ref):
  dp = degp_ref[...]
  deg = jnp.sum(dp, axis=2)
  norm = jnp.where(deg > 0, lax.rsqrt(jnp.maximum(deg, 1.0)), 0.0)
  xb = x_ref[...]
  y_ref[0] = jnp.dot(xb * norm[0, 0][:, None], w0_ref[...],
                     preferred_element_type=jnp.float32)
  y_ref[1] = jnp.dot(xb * norm[1, 0][:, None], w1_ref[...],
                     preferred_element_type=jnp.float32)
  nd_ref[0] = norm[0, 1]
  nd_ref[1] = norm[1, 1]


def _matmul_kernel(x, degp, w0, w1):
  return pl.pallas_call(
      _mm_body,
      grid=(_NRB,),
      in_specs=[
          pl.BlockSpec((_RB, D), lambda i: (i, 0)),
          pl.BlockSpec((2, 2, NS, _RB), lambda i: (0, 0, 0, i)),
          pl.BlockSpec((D, D), lambda i: (0, 0)),
          pl.BlockSpec((D, D), lambda i: (0, 0)),
      ],
      out_specs=[
          pl.BlockSpec((2, _RB, D), lambda i: (0, i, 0)),
          pl.BlockSpec((2, _RB), lambda i: (0, i)),
      ],
      out_shape=[
          jax.ShapeDtypeStruct((2, N, D), jnp.float32),
          jax.ShapeDtypeStruct((2, N), jnp.float32),
      ],
  )(x, degp, w0, w1)


def _scatter_body(y_hbm, edges_hbm, agg_hbm, acc, src_sa, dst_sa, src_sb,
                  dst_sb, srcflat, dstflat, rows_a, rows_b, fill_smem, sem,
                  sem_s, sem_d, sem_z):
  c = lax.axis_index("c")
  s = lax.axis_index("s")
  src_base = (c * 2 + 0) * EPAD + s * EPT
  dst_base = (c * 2 + 1) * EPAD + s * EPT
  yoff = c * N

  lane_iota = lax.iota(jnp.int32, LANES)
  pad_src = s * LANES + lane_iota + yoff
  pad_dst = CHUNK + lane_iota

  def pass_body(p, _):
    lo = pl.multiple_of(p * CHUNK, CHUNK)
    hi = jnp.minimum(lo + CHUNK, N)

    def zzero(i, _):
      j = i // 8
      k = i % 8
      rows_a[j, pl.ds(k * LANES, LANES)] = jnp.zeros((LANES,), jnp.float32)
      return 0

    lax.fori_loop(0, 128 * 8, zzero, 0, unroll=8)
    for k in range(5):
      pltpu.async_copy(rows_a, acc.at[pl.ds(s * ZPT + k * 128, 128)],
                       sem_z)
    for k in range(5):
      pltpu.make_async_copy(rows_a, acc.at[pl.ds(s * ZPT + k * 128, 128)],
                            sem_z).wait()
    plsc.subcore_barrier()

    QR = 128

    def gidx(j):
      return y_hbm.at[srcflat.at[pl.ds(j * QR, QR)]]

    def didx(j):
      return acc.at[dstflat.at[pl.ds(j * QR, QR)]]

    def drain(nb):
      @pl.when(nb > 0)
      def _():
        pltpu.async_copy(gidx(0), rows_a, sem)

      def drain_body(j, _):
        jeven = (j & 1) == 0

        @pl.when(jeven)
        def _():
          pltpu.make_async_copy(gidx(j), rows_a, sem).wait()

          @pl.when(j + 1 < nb)
          def _():
            pltpu.async_copy(gidx(j + 1), rows_b, sem)

          pltpu.sync_copy(rows_a, didx(j), add=True)

        @pl.when(jnp.logical_not(jeven))
        def _():
          pltpu.make_async_copy(gidx(j), rows_b, sem).wait()

          @pl.when(j + 1 < nb)
          def _():
            pltpu.async_copy(gidx(j + 1), rows_a, sem)

          pltpu.sync_copy(rows_b, didx(j), add=True)

        return 0

      lax.fori_loop(0, nb, drain_body, 0)

    def stage_start(b, src_s, dst_s):
      soff = pl.multiple_of(src_base + b * STAGE_E, STAGE_E)
      doff = pl.multiple_of(dst_base + b * STAGE_E, STAGE_E)
      pltpu.async_copy(edges_hbm.at[pl.ds(soff, STAGE_E)], src_s, sem_s)
      pltpu.async_copy(edges_hbm.at[pl.ds(doff, STAGE_E)], dst_s, sem_d)

    def stage_wait(src_s, dst_s):
      pltpu.make_async_copy(edges_hbm.at[pl.ds(0, STAGE_E)], src_s,
                            sem_s).wait()
      pltpu.make_async_copy(edges_hbm.at[pl.ds(0, STAGE_E)], dst_s,
                            sem_d).wait()

    stage_start(0, src_sa, dst_sa)

    def make_append(src_s, dst_s):
      def append_body(i, fill):
        sv = src_s[pl.ds(i * LANES, LANES)]
        dv = dst_s[pl.ds(i * LANES, LANES)]
        dloc = dv - lo
        m = dloc.astype(jnp.uint32) < (hi - lo).astype(jnp.uint32)
        plsc.store_compressed(srcflat.at[pl.ds(fill, LANES)], sv + yoff,
                              mask=m)
        plsc.store_compressed(dstflat.at[pl.ds(fill, LANES)], dloc,
                              mask=m)
        return fill + plsc.all_reduce_population_count(m)[0]
      return append_body

    def stage_body(b, fill0):
      def run(src_s, dst_s, src_n, dst_n):
        stage_wait(src_s, dst_s)

        @pl.when(b + 1 < STAGE_BLOCKS)
        def _():
          stage_start(b + 1, src_n, dst_n)

        fill_smem[0] = lax.fori_loop(0, STAGE_E // LANES,
                                     make_append(src_s, dst_s), fill0,
                                     unroll=8)

      beven = (b & 1) == 0

      @pl.when(beven)
      def _():
        run(src_sa, dst_sa, src_sb, dst_sb)

      @pl.when(jnp.logical_not(beven))
      def _():
        run(src_sb, dst_sb, src_sa, dst_sa)

      fill = fill_smem[0]
      nb = fill // QR
      drain(nb)

      @pl.when(nb > 0)
      def _():
        off = pl.multiple_of(nb * QR, QR)
        for k in range(8):
          srcflat[pl.ds(k * LANES, LANES)] = (
              srcflat[pl.ds(off + k * LANES, LANES)])
          dstflat[pl.ds(k * LANES, LANES)] = (
              dstflat[pl.ds(off + k * LANES, LANES)])

      return fill - nb * QR

    fill = lax.fori_loop(0, STAGE_BLOCKS, stage_body, 0)

    @pl.when(fill > 0)
    def _():
      f = fill
      npad = QR - f
      for k in range(8):
        cnt_k = jnp.clip(npad - k * LANES, 0, LANES)
        pm = lane_iota < cnt_k
        plsc.store_compressed(srcflat.at[pl.ds(f, LANES)], pad_src,
                              mask=pm)
        plsc.store_compressed(dstflat.at[pl.ds(f, LANES)], pad_dst,
                              mask=pm)
        f = f + cnt_k
      drain(1)

    plsc.subcore_barrier()

    rows_pt = CHUNK // NS
    foff = pl.multiple_of(c * NROW_PAD + lo + s * rows_pt, 8)
    pltpu.sync_copy(acc.at[pl.ds(s * rows_pt, rows_pt)],
                    agg_hbm.at[pl.ds(foff, rows_pt)])
    plsc.subcore_barrier()
    return 0

  lax.fori_loop(0, NPASS, pass_body, 0)


def _scatter_kernel(y2, edges1d):
  return pl.kernel(
      _scatter_body,
      out_type=jax.ShapeDtypeStruct((2 * NROW_PAD, D), jnp.float32),
      mesh=_sc_mesh(),
      scratch_types=[
          pltpu.VMEM_SHARED((ACC_ROWS, D), jnp.float32),
          pltpu.VMEM((STAGE_E,), jnp.int32),
          pltpu.VMEM((STAGE_E,), jnp.int32),
          pltpu.VMEM((STAGE_E,), jnp.int32),
          pltpu.VMEM((STAGE_E,), jnp.int32),
          pltpu.VMEM((STAGE_E + 256,), jnp.int32),
          pltpu.VMEM((STAGE_E + 256,), jnp.int32),
          pltpu.VMEM((128, D), jnp.float32),
          pltpu.VMEM((128, D), jnp.float32),
          pltpu.SMEM((1,), jnp.int32),
          pltpu.SemaphoreType.DMA,
          pltpu.SemaphoreType.DMA,
          pltpu.SemaphoreType.DMA,
          pltpu.SemaphoreType.DMA,
      ],
      compiler_params=pltpu.CompilerParams(needs_layout_passes=False),
  )(y2, edges1d)


def _post_body(a0_ref, a1_ref, nd_ref, g_ref, b_ref, out_ref, sums_ref):
  t = pl.program_id(0)
  i = pl.program_id(1)
  nd = nd_ref[...]
  hb = (a0_ref[0] * nd[0][:, None] + a1_ref[0] * nd[1][:, None])

  @pl.when((t == 0) & (i == 0))
  def _():
    sums_ref[...] = jnp.zeros_like(sums_ref)

  @pl.when(t == 0)
  def _():
    rows = i * _RB + lax.broadcasted_iota(jnp.int32, (_RB, 1), 0)
    hv = jnp.where(rows < N, hb, 0.0)
    sums_ref[0] += jnp.sum(hv, axis=0)
    sums_ref[1] += jnp.sum(hv * hv, axis=0)

  @pl.when(t == 1)
  def _():
    mean = sums_ref[0] / N
    var = jnp.maximum(sums_ref[1] / N - mean * mean, 0.0)
    scale = g_ref[0] * lax.rsqrt(var + 1e-5)
    shift = b_ref[0] - mean * scale
    tv = hb * scale + shift
    nrm = jnp.sqrt(jnp.sum(tv * tv, axis=1, keepdims=True))
    out_ref[...] = tv / jnp.maximum(nrm, 1e-12)


def _post_kernel(agg3, nd, gamma, beta):
  return pl.pallas_call(
      _post_body,
      grid=(2, _NRB),
      in_specs=[
          pl.BlockSpec((1, _RB, D), lambda t, i: (0, i, 0)),
          pl.BlockSpec((1, _RB, D), lambda t, i: (1, i, 0)),
          pl.BlockSpec((2, _RB), lambda t, i: (0, i)),
          pl.BlockSpec((1, D), lambda t, i: (0, 0)),
          pl.BlockSpec((1, D), lambda t, i: (0, 0)),
      ],
      out_specs=pl.BlockSpec((_RB, D), lambda t, i: (i, 0)),
      out_shape=jax.ShapeDtypeStruct((N, D), jnp.float32),
      scratch_shapes=[pltpu.VMEM((2, D), jnp.float32)],
  )(agg3, agg3, nd, gamma, beta)


@jax.jit
def kernel(x, edge_index_rel0, edge_index_rel1, W0, W1, gamma, beta):
  e0 = edge_index_rel0.astype(jnp.int32)
  e1 = edge_index_rel1.astype(jnp.int32)
  pad = EPAD - E
  padvals = N + (jnp.arange(pad, dtype=jnp.int32) % LANES)
  parts = []
  for ei in (e0, e1):
    for kind in range(2):
      parts.append(jnp.concatenate([ei[kind], padvals]))
  edges1d = jnp.concatenate(parts)

  degp = _degree_kernel(e0.reshape(-1), e1.reshape(-1))
  y, nd = _matmul_kernel(x, degp, W0, W1)
  y2 = y.reshape(2 * N, D)
  agg2 = _scatter_kernel(y2, edges1d)
  return _post_kernel(agg2.reshape(2, NROW_PAD, D), nd,
                      gamma.reshape(1, D), beta.reshape(1, D))


# speedup vs baseline: 1.1522x; 1.0005x over previous
import jax
import jax.numpy as jnp
from jax import lax
from jax.experimental import pallas as pl
from jax.experimental.pallas import tpu as pltpu
from jax.experimental.pallas import tpu_sc as plsc

N = 50000
D = 128
E = 400000

NC = 2
NS = 16
LANES = 16

EPT = 25600
EPAD = NS * EPT
STAGE_E = 2560
STAGE_BLOCKS = EPT // STAGE_E
NBLK = STAGE_E // 128

CHUNK = 10112
NPASS = 5
ACC_ROWS = CHUNK + 128
ZPT = ACC_ROWS // NS
NROW_PAD = NPASS * CHUNK

NP = N + 16


def _sc_mesh():
  return plsc.VectorSubcoreMesh(core_axis_name="c", subcore_axis_name="s")


EPT_R = E // NS
_DEG_FULL = EPT_R // LANES


def _deg_body(e0_hbm, e1_hbm, out_hbm, idx_v, hist_v):
  c = lax.axis_index("c")
  s = lax.axis_index("s")
  ones16 = jnp.ones((LANES,), jnp.float32)
  tailm = lax.iota(jnp.int32, LANES) < (EPT_R - _DEG_FULL * LANES)

  def process(e_ref):
    for kind in range(2):
      def zero_body(i, _):
        hist_v[pl.ds(i * LANES, LANES)] = jnp.zeros((LANES,), jnp.float32)
        return 0

      lax.fori_loop(0, NP // LANES, zero_body, 0, unroll=8)

      pltpu.sync_copy(e_ref.at[pl.ds(kind * E + s * EPT_R, EPT_R)],
                      idx_v.at[pl.ds(0, EPT_R)])

      def hist_body(i, _):
        v = idx_v[pl.ds(i * LANES, LANES)]
        plsc.addupdate_scatter(hist_v, [v], ones16)
        return 0

      lax.fori_loop(0, _DEG_FULL, hist_body, 0, unroll=8)
      vt = idx_v[pl.ds(_DEG_FULL * LANES, LANES)]
      plsc.addupdate_scatter(hist_v, [vt], ones16, mask=tailm)

      pltpu.sync_copy(hist_v, out_hbm.at[c, kind, s])

  @pl.when(c == 0)
  def _():
    process(e0_hbm)

  @pl.when(c == 1)
  def _():
    process(e1_hbm)


def _degree_kernel(e0, e1):
  return pl.kernel(
      _deg_body,
      out_type=jax.ShapeDtypeStruct((2, 2, NS, NP), jnp.float32),
      mesh=_sc_mesh(),
      scratch_types=[
          pltpu.VMEM((EPT_R + LANES,), jnp.int32),
          pltpu.VMEM((NP,), jnp.float32),
      ],
      compiler_params=pltpu.CompilerParams(needs_layout_passes=False),
  )(e0, e1)


_RB = 1024
_NRB = (N + _RB - 1) // _RB


def _mm_body(x_ref, degp_ref, w0_ref, w1_ref, y_ref, nd_ref):
  dp = degp_ref[...]
  deg = jnp.sum(dp, axis=2)
  norm = jnp.where(deg > 0, lax.rsqrt(jnp.maximum(deg, 1.0)), 0.0)
  xb = x_ref[...]
  y_ref[0] = jnp.dot(xb * norm[0, 0][:, None], w0_ref[...],
                     preferred_element_type=jnp.float32)
  y_ref[1] = jnp.dot(xb * norm[1, 0][:, None], w1_ref[...],
                     preferred_element_type=jnp.float32)
  nd_ref[0] = norm[0, 1]
  nd_ref[1] = norm[1, 1]


def _matmul_kernel(x, degp, w0, w1):
  return pl.pallas_call(
      _mm_body,
      grid=(_NRB,),
      in_specs=[
          pl.BlockSpec((_RB, D), lambda i: (i, 0)),
          pl.BlockSpec((2, 2, NS, _RB), lambda i: (0, 0, 0, i)),
          pl.BlockSpec((D, D), lambda i: (0, 0)),
          pl.BlockSpec((D, D), lambda i: (0, 0)),
      ],
      out_specs=[
          pl.BlockSpec((2, _RB, D), lambda i: (0, i, 0)),
          pl.BlockSpec((2, _RB), lambda i: (0, i)),
      ],
      out_shape=[
          jax.ShapeDtypeStruct((2, N, D), jnp.float32),
          jax.ShapeDtypeStruct((2, N), jnp.float32),
      ],
  )(x, degp, w0, w1)


def _scatter_body(y_hbm, edges_hbm, agg_hbm, acc, src_sa, dst_sa, src_sb,
                  dst_sb, srcflat, dstflat, rows_a, rows_b, fill_smem, sem,
                  sem_s, sem_d, sem_z):
  c = lax.axis_index("c")
  s = lax.axis_index("s")
  src_base = (c * 2 + 0) * EPAD + s * EPT
  dst_base = (c * 2 + 1) * EPAD + s * EPT
  yoff = c * N

  lane_iota = lax.iota(jnp.int32, LANES)
  pad_src = s * LANES + lane_iota + yoff
  pad_dst = CHUNK + lane_iota

  def pass_body(p, _):
    lo = pl.multiple_of(p * CHUNK, CHUNK)
    hi = jnp.minimum(lo + CHUNK, N)

    def zzero(i, _):
      j = i // 8
      k = i % 8
      rows_a[j, pl.ds(k * LANES, LANES)] = jnp.zeros((LANES,), jnp.float32)
      return 0

    lax.fori_loop(0, 128 * 8, zzero, 0, unroll=8)
    for k in range(5):
      pltpu.async_copy(rows_a, acc.at[pl.ds(s * ZPT + k * 128, 128)],
                       sem_z)
    for k in range(5):
      pltpu.make_async_copy(rows_a, acc.at[pl.ds(s * ZPT + k * 128, 128)],
                            sem_z).wait()
    plsc.subcore_barrier()

    QR = 128

    def gidx(j):
      return y_hbm.at[srcflat.at[pl.ds(j * QR, QR)]]

    def didx(j):
      return acc.at[dstflat.at[pl.ds(j * QR, QR)]]

    def drain(nb):
      @pl.when(nb > 0)
      def _():
        pltpu.async_copy(gidx(0), rows_a, sem)

      def drain_body(j, _):
        jeven = (j & 1) == 0

        @pl.when(jeven)
        def _():
          pltpu.make_async_copy(gidx(j), rows_a, sem).wait()

          @pl.when(j + 1 < nb)
          def _():
            pltpu.async_copy(gidx(j + 1), rows_b, sem)

          pltpu.sync_copy(rows_a, didx(j), add=True)

        @pl.when(jnp.logical_not(jeven))
        def _():
          pltpu.make_async_copy(gidx(j), rows_b, sem).wait()

          @pl.when(j + 1 < nb)
          def _():
            pltpu.async_copy(gidx(j + 1), rows_a, sem)

          pltpu.sync_copy(rows_b, didx(j), add=True)

        return 0

      lax.fori_loop(0, nb, drain_body, 0)

    def stage_start(b, src_s, dst_s):
      soff = pl.multiple_of(src_base + b * STAGE_E, STAGE_E)
      doff = pl.multiple_of(dst_base + b * STAGE_E, STAGE_E)
      pltpu.async_copy(edges_hbm.at[pl.ds(soff, STAGE_E)], src_s, sem_s)
      pltpu.async_copy(edges_hbm.at[pl.ds(doff, STAGE_E)], dst_s, sem_d)

    def stage_wait(src_s, dst_s):
      pltpu.make_async_copy(edges_hbm.at[pl.ds(0, STAGE_E)], src_s,
                            sem_s).wait()
      pltpu.make_async_copy(edges_hbm.at[pl.ds(0, STAGE_E)], dst_s,
                            sem_d).wait()

    stage_start(0, src_sa, dst_sa)

    def make_append(src_s, dst_s):
      def append_body(i, fill):
        sv = src_s[pl.ds(i * LANES, LANES)]
        dv = dst_s[pl.ds(i * LANES, LANES)]
        dloc = dv - lo
        m = dloc.astype(jnp.uint32) < (hi - lo).astype(jnp.uint32)
        plsc.store_compressed(srcflat.at[pl.ds(fill, LANES)], sv + yoff,
                              mask=m)
        plsc.store_compressed(dstflat.at[pl.ds(fill, LANES)], dloc,
                              mask=m)
        return fill + plsc.all_reduce_population_count(m)[0]
      return append_body

    def stage_body(b, fill0):
      def run(src_s, dst_s, src_n, dst_n):
        stage_wait(src_s, dst_s)

        @pl.when(b + 1 < STAGE_BLOCKS)
        def _():
          stage_start(b + 1, src_n, dst_n)

        fill_smem[0] = lax.fori_loop(0, STAGE_E // LANES,
                                     make_append(src_s, dst_s), fill0,
                                     unroll=8)

      beven = (b & 1) == 0

      @pl.when(beven)
      def _():
        run(src_sa, dst_sa, src_sb, dst_sb)

      @pl.when(jnp.logical_not(beven))
      def _():
        run(src_sb, dst_sb, src_sa, dst_sa)

      fill = fill_smem[0]
      nb = fill // QR
      drain(nb)

      @pl.when(nb > 0)
      def _():
        off = pl.multiple_of(nb * QR, QR)
        for k in range(8):
          srcflat[pl.ds(k * LANES, LANES)] = (
              srcflat[pl.ds(off + k * LANES, LANES)])
          dstflat[pl.ds(k * LANES, LANES)] = (
              dstflat[pl.ds(off + k * LANES, LANES)])

      return fill - nb * QR

    fill = lax.fori_loop(0, STAGE_BLOCKS, stage_body, 0)

    @pl.when(fill > 0)
    def _():
      f = fill
      npad = QR - f
      for k in range(8):
        cnt_k = jnp.clip(npad - k * LANES, 0, LANES)
        pm = lane_iota < cnt_k
        plsc.store_compressed(srcflat.at[pl.ds(f, LANES)], pad_src,
                              mask=pm)
        plsc.store_compressed(dstflat.at[pl.ds(f, LANES)], pad_dst,
                              mask=pm)
        f = f + cnt_k
      drain(1)

    plsc.subcore_barrier()

    rows_pt = CHUNK // NS
    foff = pl.multiple_of(c * NROW_PAD + lo + s * rows_pt, 8)
    pltpu.sync_copy(acc.at[pl.ds(s * rows_pt, rows_pt)],
                    agg_hbm.at[pl.ds(foff, rows_pt)])
    plsc.subcore_barrier()
    return 0

  lax.fori_loop(0, NPASS, pass_body, 0)


def _scatter_kernel(y2, edges1d):
  return pl.kernel(
      _scatter_body,
      out_type=jax.ShapeDtypeStruct((2 * NROW_PAD, D), jnp.float32),
      mesh=_sc_mesh(),
      scratch_types=[
          pltpu.VMEM_SHARED((ACC_ROWS, D), jnp.float32),
          pltpu.VMEM((STAGE_E,), jnp.int32),
          pltpu.VMEM((STAGE_E,), jnp.int32),
          pltpu.VMEM((STAGE_E,), jnp.int32),
          pltpu.VMEM((STAGE_E,), jnp.int32),
          pltpu.VMEM((STAGE_E + 256,), jnp.int32),
          pltpu.VMEM((STAGE_E + 256,), jnp.int32),
          pltpu.VMEM((128, D), jnp.float32),
          pltpu.VMEM((128, D), jnp.float32),
          pltpu.SMEM((1,), jnp.int32),
          pltpu.SemaphoreType.DMA,
          pltpu.SemaphoreType.DMA,
          pltpu.SemaphoreType.DMA,
          pltpu.SemaphoreType.DMA,
      ],
      compiler_params=pltpu.CompilerParams(needs_layout_passes=False),
  )(y2, edges1d)


def _post_body(a0_ref, a1_ref, nd_ref, g_ref, b_ref, out_ref, sums_ref):
  t = pl.program_id(0)
  i = pl.program_id(1)
  nd = nd_ref[...]
  hb = (a0_ref[0] * nd[0][:, None] + a1_ref[0] * nd[1][:, None])

  @pl.when((t == 0) & (i == 0))
  def _():
    sums_ref[...] = jnp.zeros_like(sums_ref)

  @pl.when(t == 0)
  def _():
    rows = i * _RB + lax.broadcasted_iota(jnp.int32, (_RB, 1), 0)
    hv = jnp.where(rows < N, hb, 0.0)
    sums_ref[0] += jnp.sum(hv, axis=0)
    sums_ref[1] += jnp.sum(hv * hv, axis=0)

  @pl.when(t == 1)
  def _():
    mean = sums_ref[0] / N
    var = jnp.maximum(sums_ref[1] / N - mean * mean, 0.0)
    scale = g_ref[0] * lax.rsqrt(var + 1e-5)
    shift = b_ref[0] - mean * scale
    tv = hb * scale + shift
    nrm = jnp.sqrt(jnp.sum(tv * tv, axis=1, keepdims=True))
    out_ref[...] = tv / jnp.maximum(nrm, 1e-12)


def _post_kernel(agg3, nd, gamma, beta):
  return pl.pallas_call(
      _post_body,
      grid=(2, _NRB),
      in_specs=[
          pl.BlockSpec((1, _RB, D), lambda t, i: (0, i, 0)),
          pl.BlockSpec((1, _RB, D), lambda t, i: (1, i, 0)),
          pl.BlockSpec((2, _RB), lambda t, i: (0, i)),
          pl.BlockSpec((1, D), lambda t, i: (0, 0)),
          pl.BlockSpec((1, D), lambda t, i: (0, 0)),
      ],
      out_specs=pl.BlockSpec((_RB, D), lambda t, i: (i, 0)),
      out_shape=jax.ShapeDtypeStruct((N, D), jnp.float32),
      scratch_shapes=[pltpu.VMEM((2, D), jnp.float32)],
  )(agg3, agg3, nd, gamma, beta)


@jax.jit
def kernel(x, edge_index_rel0, edge_index_rel1, W0, W1, gamma, beta):
  e0 = edge_index_rel0.astype(jnp.int32)
  e1 = edge_index_rel1.astype(jnp.int32)
  pad = EPAD - E
  padvals = N + (jnp.arange(pad, dtype=jnp.int32) % LANES)
  parts = []
  for ei in (e0, e1):
    for kind in range(2):
      parts.append(jnp.concatenate([ei[kind], padvals]))
  edges1d = jnp.concatenate(parts)

  degp = _degree_kernel(e0.reshape(-1), e1.reshape(-1))
  y, nd = _matmul_kernel(x, degp, W0, W1)
  y2 = y.reshape(2 * N, D)
  agg2 = _scatter_kernel(y2, edges1d)
  return _post_kernel(agg2.reshape(2, NROW_PAD, D), nd,
                      gamma.reshape(1, D), beta.reshape(1, D))
